# baseline (speedup 1.0000x reference)
import functools

import jax
import jax.numpy as jnp
from jax import lax
from jax.experimental import pallas as pl
from jax.experimental.pallas import tpu as pltpu
from jax.experimental.pallas import tpu_sc as plsc

B = 50000
N_NODES = 50000
D = 128
S = 10

NC = 2
NS = 16
NW = NC * NS
CB = 32
NCHUNK = 50
BPW = CB * NCHUNK
BP = NW * BPW
LANES = 16

_sc_mesh = plsc.VectorSubcoreMesh(core_axis_name="c", subcore_axis_name="s")


@functools.partial(
    pl.kernel,
    out_type=[
        jax.ShapeDtypeStruct((BP, D), jnp.float32),
        jax.ShapeDtypeStruct((BP, D), jnp.float32),
    ],
    mesh=_sc_mesh,
    compiler_params=pltpu.CompilerParams(use_tc_tiling_on_sc=True),
    scratch_types=[
        pltpu.VMEM((CB,), jnp.int32),
        pltpu.VMEM((CB,), jnp.int32),
        pltpu.VMEM((CB * S,), jnp.int32),
        pltpu.VMEM((CB * S,), jnp.int32),
        pltpu.VMEM((CB, D), jnp.float32),
        pltpu.VMEM((CB, D), jnp.float32),
        pltpu.VMEM((CB * S, D), jnp.float32),
        pltpu.VMEM((CB * S, D), jnp.float32),
        pltpu.VMEM((CB, D), jnp.float32),
        pltpu.VMEM((CB, D), jnp.float32),
        pltpu.SemaphoreType.DMA,
        pltpu.SemaphoreType.DMA,
        pltpu.SemaphoreType.DMA,
        pltpu.SemaphoreType.DMA,
        pltpu.SemaphoreType.DMA,
        pltpu.SemaphoreType.DMA,
        pltpu.SemaphoreType.DMA,
        pltpu.SemaphoreType.DMA,
    ],
)
def _gather_mean(nodes_hbm, nidx_hbm, feat_hbm, selfo_hbm, neigho_hbm,
                 idxs_v0, idxs_v1, idxn_v0, idxn_v1,
                 selfb_v0, selfb_v1, rows_v0, rows_v1, acc_v0, acc_v1,
                 sem_s0, sem_s1, sem_n0, sem_n1,
                 sem_ws0, sem_ws1, sem_wa0, sem_wa1):
    wid = lax.axis_index("s") * NC + lax.axis_index("c")
    base_w = wid * BPW
    idxs_v = (idxs_v0, idxs_v1)
    idxn_v = (idxn_v0, idxn_v1)
    selfb_v = (selfb_v0, selfb_v1)
    rows_v = (rows_v0, rows_v1)
    acc_v = (acc_v0, acc_v1)
    sem_s = (sem_s0, sem_s1)
    sem_n = (sem_n0, sem_n1)
    sem_ws = (sem_ws0, sem_ws1)
    sem_wa = (sem_wa0, sem_wa1)

    def issue_gathers(c, p):
        base = base_w + c * CB
        pltpu.sync_copy(nodes_hbm.at[pl.ds(base, CB)], idxs_v[p])
        pltpu.sync_copy(nidx_hbm.at[pl.ds(base * S, CB * S)], idxn_v[p])
        pltpu.async_copy(feat_hbm.at[idxs_v[p]], selfb_v[p], sem_s[p])
        pltpu.async_copy(feat_hbm.at[idxn_v[p]], rows_v[p], sem_n[p])

    def wait_gathers(p):
        pltpu.make_async_copy(
            feat_hbm.at[pl.ds(0, CB)], selfb_v[p], sem_s[p]).wait()
        pltpu.make_async_copy(
            feat_hbm.at[pl.ds(0, CB * S)], rows_v[p], sem_n[p]).wait()

    def issue_self_wb(c, p):
        pltpu.async_copy(
            selfb_v[p], selfo_hbm.at[pl.ds(base_w + c * CB, CB)],
            sem_ws[p])

    def wait_self_wb(p):
        pltpu.make_async_copy(
            selfb_v[p], selfo_hbm.at[pl.ds(0, CB)], sem_ws[p]).wait()

    def issue_acc_wb(c, p):
        pltpu.async_copy(
            acc_v[p], neigho_hbm.at[pl.ds(base_w + c * CB, CB)],
            sem_wa[p])

    def wait_acc_wb(p):
        pltpu.make_async_copy(
            acc_v[p], neigho_hbm.at[pl.ds(0, CB)], sem_wa[p]).wait()

    def accumulate(p):
        def row_body(b, carry):
            r0 = b * S
            for j in range(D // LANES):
                col = pl.ds(j * LANES, LANES)
                a = rows_v[p][r0, col]
                for s in range(1, S):
                    a = a + rows_v[p][r0 + s, col]
                acc_v[p][b, col] = a
            return carry
        lax.fori_loop(0, CB, row_body, 0)

    issue_gathers(0, 0)

    def chunk_body(c, p):
        q = 1 - p
        pl.when(c >= 1)(lambda: wait_self_wb(q))
        pl.when(c + 1 < NCHUNK)(lambda: issue_gathers(c + 1, q))
        wait_gathers(p)
        issue_self_wb(c, p)
        pl.when(c >= 2)(lambda: wait_acc_wb(p))
        accumulate(p)
        issue_acc_wb(c, p)

    def pair_body(g, carry):
        chunk_body(2 * g, 0)
        chunk_body(2 * g + 1, 1)
        return carry

    lax.fori_loop(0, NCHUNK // 2, pair_body, 0)

    wait_self_wb(1)
    wait_acc_wb(0)
    wait_acc_wb(1)


TB = 1024


def _proj_body(w1_ref, w2_ref, s_ref, n_ref, o_ref):
    a = lax.dot_general(w1_ref[...], s_ref[...], (((1,), (1,)), ((), ())),
                        preferred_element_type=jnp.float32)
    b = lax.dot_general(w2_ref[...], n_ref[...], (((1,), (1,)), ((), ())),
                        preferred_element_type=jnp.float32)
    o_ref[...] = jnp.maximum(a + b, 0.0)


_proj = pl.pallas_call(
    _proj_body,
    grid=(BP // TB,),
    in_specs=[
        pl.BlockSpec((D, D), lambda i: (0, 0)),
        pl.BlockSpec((D, D), lambda i: (0, 0)),
        pl.BlockSpec((TB, D), lambda i: (i, 0)),
        pl.BlockSpec((TB, D), lambda i: (i, 0)),
    ],
    out_specs=pl.BlockSpec((D, TB), lambda i: (0, i)),
    out_shape=jax.ShapeDtypeStruct((D, BP), jnp.float32),
)


def kernel(nodes, neigh_idx, feat_data, W):
    nodes = nodes.astype(jnp.int32)
    neigh_idx = neigh_idx.astype(jnp.int32)
    pad = BP - B
    nodes_p = jnp.concatenate([nodes, jnp.zeros((pad,), jnp.int32)])
    nidx_p = jnp.concatenate(
        [neigh_idx, jnp.zeros((pad, S), jnp.int32)]).reshape(-1)
    self_g, neigh_sum = _gather_mean(nodes_p, nidx_p, feat_data)
    w1 = W[:, :D]
    w2 = W[:, D:] * (1.0 / S)
    out = _proj(w1, w2, self_g, neigh_sum)
    return out[:, :B]


# speedup vs baseline: 1.0262x; 1.0262x over previous
import functools

import jax
import jax.numpy as jnp
from jax import lax
from jax.experimental import pallas as pl
from jax.experimental.pallas import tpu as pltpu
from jax.experimental.pallas import tpu_sc as plsc

B = 50000
N_NODES = 50000
D = 128
S = 10

NC = 2
NS = 16
NW = NC * NS
CB = 32
NCHUNK = 50
BPW = CB * NCHUNK
BP = NW * BPW
LANES = 16

_sc_mesh = plsc.VectorSubcoreMesh(core_axis_name="c", subcore_axis_name="s")


@functools.partial(
    pl.kernel,
    out_type=[
        jax.ShapeDtypeStruct((BP, D), jnp.float32),
        jax.ShapeDtypeStruct((BP, D), jnp.float32),
    ],
    mesh=_sc_mesh,
    compiler_params=pltpu.CompilerParams(use_tc_tiling_on_sc=True),
    scratch_types=[
        pltpu.VMEM((CB,), jnp.int32),
        pltpu.VMEM((CB,), jnp.int32),
        pltpu.VMEM((CB * S,), jnp.int32),
        pltpu.VMEM((CB * S,), jnp.int32),
        pltpu.VMEM((CB, D), jnp.float32),
        pltpu.VMEM((CB, D), jnp.float32),
        pltpu.VMEM((CB * S, D), jnp.float32),
        pltpu.VMEM((CB * S, D), jnp.float32),
        pltpu.VMEM((CB, D), jnp.float32),
        pltpu.VMEM((CB, D), jnp.float32),
        pltpu.SemaphoreType.DMA,
        pltpu.SemaphoreType.DMA,
        pltpu.SemaphoreType.DMA,
        pltpu.SemaphoreType.DMA,
        pltpu.SemaphoreType.DMA,
        pltpu.SemaphoreType.DMA,
        pltpu.SemaphoreType.DMA,
        pltpu.SemaphoreType.DMA,
    ],
)
def _gather_mean(nodes_hbm, nidx_hbm, feat_hbm, selfo_hbm, neigho_hbm,
                 idxs_v0, idxs_v1, idxn_v0, idxn_v1,
                 selfb_v0, selfb_v1, rows_v0, rows_v1, acc_v0, acc_v1,
                 sem_s0, sem_s1, sem_n0, sem_n1,
                 sem_ws0, sem_ws1, sem_wa0, sem_wa1):
    wid = lax.axis_index("s") * NC + lax.axis_index("c")
    base_w = wid * BPW
    idxs_v = (idxs_v0, idxs_v1)
    idxn_v = (idxn_v0, idxn_v1)
    selfb_v = (selfb_v0, selfb_v1)
    rows_v = (rows_v0, rows_v1)
    acc_v = (acc_v0, acc_v1)
    sem_s = (sem_s0, sem_s1)
    sem_n = (sem_n0, sem_n1)
    sem_ws = (sem_ws0, sem_ws1)
    sem_wa = (sem_wa0, sem_wa1)

    def issue_gathers(c, p):
        base = base_w + c * CB
        pltpu.sync_copy(nodes_hbm.at[pl.ds(base, CB)], idxs_v[p])
        pltpu.sync_copy(nidx_hbm.at[pl.ds(base * S, CB * S)], idxn_v[p])
        pltpu.async_copy(feat_hbm.at[idxs_v[p]], selfb_v[p], sem_s[p])
        pltpu.async_copy(feat_hbm.at[idxn_v[p]], rows_v[p], sem_n[p])

    def wait_gathers(p):
        pltpu.make_async_copy(
            feat_hbm.at[pl.ds(0, CB)], selfb_v[p], sem_s[p]).wait()
        pltpu.make_async_copy(
            feat_hbm.at[pl.ds(0, CB * S)], rows_v[p], sem_n[p]).wait()

    def issue_self_wb(c, p):
        pltpu.async_copy(
            selfb_v[p], selfo_hbm.at[pl.ds(base_w + c * CB, CB)],
            sem_ws[p])

    def wait_self_wb(p):
        pltpu.make_async_copy(
            selfb_v[p], selfo_hbm.at[pl.ds(0, CB)], sem_ws[p]).wait()

    def issue_acc_wb(c, p):
        pltpu.async_copy(
            acc_v[p], neigho_hbm.at[pl.ds(base_w + c * CB, CB)],
            sem_wa[p])

    def wait_acc_wb(p):
        pltpu.make_async_copy(
            acc_v[p], neigho_hbm.at[pl.ds(0, CB)], sem_wa[p]).wait()

    def accumulate(p):
        def row_body(b, carry):
            r0 = b * S
            for j in range(D // LANES):
                col = pl.ds(j * LANES, LANES)
                a = rows_v[p][r0, col]
                for s in range(1, S):
                    a = a + rows_v[p][r0 + s, col]
                acc_v[p][b, col] = a
            return carry
        lax.fori_loop(0, CB, row_body, 0)

    issue_gathers(0, 0)

    def chunk_body(c, p):
        q = 1 - p
        pl.when(c >= 1)(lambda: wait_self_wb(q))
        pl.when(c + 1 < NCHUNK)(lambda: issue_gathers(c + 1, q))
        wait_gathers(p)
        issue_self_wb(c, p)
        pl.when(c >= 2)(lambda: wait_acc_wb(p))
        accumulate(p)
        issue_acc_wb(c, p)

    def pair_body(g, carry):
        chunk_body(2 * g, 0)
        chunk_body(2 * g + 1, 1)
        return carry

    lax.fori_loop(0, NCHUNK // 2, pair_body, 0)

    wait_self_wb(1)
    wait_acc_wb(0)
    wait_acc_wb(1)


TB = 1024


def _proj_body(w1_ref, w2_ref, s_ref, n_ref, o_ref):
    a = lax.dot_general(s_ref[...], w1_ref[...], (((1,), (1,)), ((), ())),
                        preferred_element_type=jnp.float32)
    b = lax.dot_general(n_ref[...], w2_ref[...], (((1,), (1,)), ((), ())),
                        preferred_element_type=jnp.float32)
    o_ref[...] = jnp.maximum(a + b, 0.0)


_proj = pl.pallas_call(
    _proj_body,
    grid=(BP // TB,),
    in_specs=[
        pl.BlockSpec((D, D), lambda i: (0, 0)),
        pl.BlockSpec((D, D), lambda i: (0, 0)),
        pl.BlockSpec((TB, D), lambda i: (i, 0)),
        pl.BlockSpec((TB, D), lambda i: (i, 0)),
    ],
    out_specs=pl.BlockSpec((TB, D), lambda i: (i, 0)),
    out_shape=jax.ShapeDtypeStruct((BP, D), jnp.float32),
)


def kernel(nodes, neigh_idx, feat_data, W):
    nodes = nodes.astype(jnp.int32)
    neigh_idx = neigh_idx.astype(jnp.int32)
    pad = BP - B
    nodes_p = jnp.concatenate([nodes, jnp.zeros((pad,), jnp.int32)])
    nidx_p = jnp.concatenate(
        [neigh_idx, jnp.zeros((pad, S), jnp.int32)]).reshape(-1)
    self_g, neigh_sum = _gather_mean(nodes_p, nidx_p, feat_data)
    w1 = W[:, :D]
    w2 = W[:, D:] * (1.0 / S)
    out_t = _proj(w1, w2, self_g, neigh_sum)
    return out_t[:B].T


# speedup vs baseline: 1.5342x; 1.4950x over previous
import functools

import jax
import jax.numpy as jnp
from jax import lax
from jax.experimental import pallas as pl
from jax.experimental.pallas import tpu as pltpu
from jax.experimental.pallas import tpu_sc as plsc

B = 50000
N_NODES = 50000
D = 128
S = 10

NC = 2
NS = 16
NW = NC * NS
CB = 32
NCHUNK = 49
BPW = CB * NCHUNK
BP = NW * BPW
LANES = 16

_sc_mesh = plsc.VectorSubcoreMesh(core_axis_name="c", subcore_axis_name="s")


@functools.partial(
    pl.kernel,
    out_type=[
        jax.ShapeDtypeStruct((BP, D), jnp.float32),
        jax.ShapeDtypeStruct((BP, D), jnp.float32),
    ],
    mesh=_sc_mesh,
    scratch_types=[
        pltpu.VMEM((CB,), jnp.int32),
        pltpu.VMEM((CB * S,), jnp.int32),
        pltpu.VMEM((CB, D), jnp.float32),
        pltpu.VMEM((CB * S, D), jnp.float32),
        pltpu.VMEM((CB, D), jnp.float32),
        pltpu.SemaphoreType.DMA,
        pltpu.SemaphoreType.DMA,
    ],
)
def _gather_mean(nodes_hbm, nidx_hbm, feat_hbm, selfo_hbm, neigho_hbm,
                 idxs_v, idxn_v, selfb_v, rows_v, acc_v, sem_s, sem_n):
    wid = lax.axis_index("s") * NC + lax.axis_index("c")
    base_w = wid * BPW

    def chunk_body(c, carry):
        base = base_w + c * CB
        pltpu.sync_copy(nodes_hbm.at[pl.ds(base, CB)], idxs_v)
        pltpu.sync_copy(nidx_hbm.at[pl.ds(base * S, CB * S)], idxn_v)
        cp_s = pltpu.async_copy(feat_hbm.at[idxs_v], selfb_v, sem_s)
        cp_n = pltpu.async_copy(feat_hbm.at[idxn_v], rows_v, sem_n)
        cp_s.wait()
        pltpu.sync_copy(selfb_v, selfo_hbm.at[pl.ds(base, CB)])
        cp_n.wait()

        def row_body(b, carry2):
            r0 = b * S
            for j in range(D // LANES):
                col = pl.ds(j * LANES, LANES)
                a = rows_v[r0, col]
                for s in range(1, S):
                    a = a + rows_v[r0 + s, col]
                acc_v[b, col] = a
            return carry2

        lax.fori_loop(0, CB, row_body, 0)
        pltpu.sync_copy(acc_v, neigho_hbm.at[pl.ds(base, CB)])
        return carry

    lax.fori_loop(0, NCHUNK, chunk_body, 0)


TB = 1024


def _proj_body(w1_ref, w2_ref, s_ref, n_ref, o_ref):
    a = lax.dot_general(s_ref[...], w1_ref[...], (((1,), (1,)), ((), ())),
                        preferred_element_type=jnp.float32)
    b = lax.dot_general(n_ref[...], w2_ref[...], (((1,), (1,)), ((), ())),
                        preferred_element_type=jnp.float32)
    o_ref[...] = jnp.maximum(a + b, 0.0)


_proj = pl.pallas_call(
    _proj_body,
    grid=(BP // TB,),
    in_specs=[
        pl.BlockSpec((D, D), lambda i: (0, 0)),
        pl.BlockSpec((D, D), lambda i: (0, 0)),
        pl.BlockSpec((TB, D), lambda i: (i, 0)),
        pl.BlockSpec((TB, D), lambda i: (i, 0)),
    ],
    out_specs=pl.BlockSpec((TB, D), lambda i: (i, 0)),
    out_shape=jax.ShapeDtypeStruct((BP, D), jnp.float32),
)


def kernel(nodes, neigh_idx, feat_data, W):
    nodes = nodes.astype(jnp.int32)
    neigh_idx = neigh_idx.astype(jnp.int32)
    pad = BP - B
    nodes_p = jnp.concatenate([nodes, jnp.zeros((pad,), jnp.int32)])
    nidx_p = jnp.concatenate(
        [neigh_idx, jnp.zeros((pad, S), jnp.int32)]).reshape(-1)
    self_g, neigh_sum = _gather_mean(nodes_p, nidx_p, feat_data)
    w1 = W[:, :D]
    w2 = W[:, D:] * (1.0 / S)
    out_t = _proj(w1, w2, self_g, neigh_sum)
    return out_t[:B].T
